# baseline (speedup 1.0000x reference)
import functools

import jax
import jax.numpy as jnp
from jax import lax
from jax.experimental import pallas as pl
from jax.experimental.pallas import tpu as pltpu
from jax.experimental.pallas import tpu_sc as plsc

N = 50000
CIN = 32
COUT = 32
KVOL = 27
PACK = 4
YW = PACK * COUT

NW = 32
ROWS_PER_W = 1664
NPAD = NW * ROWS_PER_W
BLK = 512
NB = NPAD // PACK // BLK


def _tc_body(x_ref, w_ref, b_ref, y_ref):
    x = x_ref[...]
    for k in range(KVOL):
        y = jnp.dot(x, w_ref[k], preferred_element_type=jnp.float32)
        if k == 0:
            y = y + b_ref[...]
        y_ref[k] = y


def _tc_gemm(x4, w4, b128):
    return pl.pallas_call(
        _tc_body,
        grid=(NB,),
        in_specs=[
            pl.BlockSpec((BLK, YW), lambda i: (i, 0)),
            pl.BlockSpec((KVOL, YW, YW), lambda i: (0, 0, 0)),
            pl.BlockSpec((1, YW), lambda i: (0, 0)),
        ],
        out_specs=pl.BlockSpec((KVOL, BLK, YW), lambda i: (0, i, 0)),
        out_shape=jax.ShapeDtypeStruct((KVOL, NPAD // PACK, YW), jnp.float32),
    )(x4, w4, b128)


def _sc_gather_sum(y, nbr1d):
    nc = 2
    mesh = plsc.VectorSubcoreMesh(
        core_axis_name="c", subcore_axis_name="s", num_cores=nc, num_subcores=16
    )

    y2d = y.reshape(KVOL * NPAD, COUT)

    @functools.partial(
        pl.kernel,
        out_type=jax.ShapeDtypeStruct((NPAD, COUT), jnp.float32),
        mesh=mesh,
        scratch_types=[
            pltpu.VMEM((KVOL * ROWS_PER_W,), jnp.int32),
            pltpu.VMEM((ROWS_PER_W, COUT), jnp.float32),
            pltpu.SemaphoreType.DMA,
            pltpu.SemaphoreType.DMA,
        ],
        compiler_params=pltpu.CompilerParams(use_tc_tiling_on_sc=False),
    )
    def sc_k(y_hbm, nbr_hbm, out_hbm, idx_v, acc_v, sem, isem):
        wid = lax.axis_index("s") * nc + lax.axis_index("c")
        base = wid * ROWS_PER_W
        idescs = [
            pltpu.async_copy(
                nbr_hbm.at[pl.ds(k * NPAD + base, ROWS_PER_W)],
                idx_v.at[pl.ds(k * ROWS_PER_W, ROWS_PER_W)],
                isem,
            )
            for k in range(KVOL)
        ]
        z = jnp.zeros((16,), jnp.float32)

        def zero_body(r, carry):
            acc_v[r, pl.ds(0, 16)] = z
            acc_v[r, pl.ds(16, 16)] = z
            return carry

        lax.fori_loop(0, ROWS_PER_W, zero_body, 0)
        for d in idescs:
            d.wait()

        descs = []
        nv = ROWS_PER_W // 16
        for k in range(KVOL):
            if k > 0:
                def off_body(j, carry, _off=k * NPAD, _base=k * nv):
                    idx_v[pl.ds((_base + j) * 16, 16)] += _off
                    return carry

                lax.fori_loop(0, nv, off_body, 0)
            descs.append(
                pltpu.async_copy(
                    y_hbm.at[idx_v.at[pl.ds(k * ROWS_PER_W, ROWS_PER_W)]],
                    acc_v,
                    sem,
                    add=True,
                )
            )
        for d in descs:
            d.wait()
        pltpu.sync_copy(acc_v, out_hbm.at[pl.ds(base, ROWS_PER_W)])

    return sc_k(y2d, nbr1d)


def kernel(in_feature, nbr_idx, weight, bias):
    w = weight.reshape(COUT, CIN, KVOL).transpose(2, 1, 0)
    w4 = jnp.einsum("tu,kcd->ktcud", jnp.eye(PACK, dtype=w.dtype), w)
    w4 = w4.reshape(KVOL, YW, YW)
    b128 = jnp.tile(bias.reshape(1, COUT), (1, PACK))
    x4 = jnp.pad(
        in_feature.reshape(N // PACK, YW), ((0, (NPAD - N) // PACK), (0, 0))
    )
    nbr1d = jnp.pad(nbr_idx, ((0, 0), (0, NPAD - N))).reshape(-1)
    y = _tc_gemm(x4, w4, b128)
    out = _sc_gather_sum(y, nbr1d)
    return out[:N]


# speedup vs baseline: 1.0110x; 1.0110x over previous
import functools

import jax
import jax.numpy as jnp
from jax import lax
from jax.experimental import pallas as pl
from jax.experimental.pallas import tpu as pltpu
from jax.experimental.pallas import tpu_sc as plsc

N = 50000
CIN = 32
COUT = 32
KVOL = 27
PACK = 4
YW = PACK * COUT

NW = 32
ROWS_PER_W = 1664
NPAD = NW * ROWS_PER_W
BLK = 512
NB = NPAD // PACK // BLK


def _tc_body(x_ref, w_ref, b_ref, y_ref):
    x = x_ref[...]
    for k in range(KVOL):
        y = jnp.dot(x, w_ref[k], preferred_element_type=jnp.float32)
        if k == 0:
            y = y + b_ref[...]
        y_ref[k] = y


def _tc_gemm(x4, w4, b128):
    return pl.pallas_call(
        _tc_body,
        grid=(NB,),
        in_specs=[
            pl.BlockSpec((BLK, YW), lambda i: (i, 0)),
            pl.BlockSpec((KVOL, YW, YW), lambda i: (0, 0, 0)),
            pl.BlockSpec((1, YW), lambda i: (0, 0)),
        ],
        out_specs=pl.BlockSpec((KVOL, BLK, YW), lambda i: (0, i, 0)),
        out_shape=jax.ShapeDtypeStruct((KVOL, NPAD // PACK, YW), jnp.float32),
    )(x4, w4, b128)


def _sc_gather_sum(y, nbr1d):
    nc = 2
    mesh = plsc.VectorSubcoreMesh(
        core_axis_name="c", subcore_axis_name="s", num_cores=nc, num_subcores=16
    )

    y2d = y.reshape(KVOL * NPAD, COUT)

    @functools.partial(
        pl.kernel,
        out_type=jax.ShapeDtypeStruct((NPAD, COUT), jnp.float32),
        mesh=mesh,
        scratch_types=[
            pltpu.VMEM((KVOL * ROWS_PER_W,), jnp.int32),
            pltpu.VMEM((ROWS_PER_W, COUT), jnp.float32),
            pltpu.SemaphoreType.DMA,
            pltpu.SemaphoreType.DMA,
        ],
        compiler_params=pltpu.CompilerParams(use_tc_tiling_on_sc=False),
    )
    def sc_k(y_hbm, nbr_hbm, out_hbm, idx_v, acc_v, sem, isem):
        wid = lax.axis_index("s") * nc + lax.axis_index("c")
        base = wid * ROWS_PER_W
        idescs = [
            pltpu.async_copy(
                nbr_hbm.at[pl.ds(k * NPAD + base, ROWS_PER_W)],
                idx_v.at[pl.ds(k * ROWS_PER_W, ROWS_PER_W)],
                isem,
            )
            for k in range(KVOL)
        ]
        z = jnp.zeros((16,), jnp.float32)

        def zero_body(r, carry):
            acc_v[r, pl.ds(0, 16)] = z
            acc_v[r, pl.ds(16, 16)] = z
            return carry

        lax.fori_loop(0, ROWS_PER_W, zero_body, 0)
        for d in idescs:
            d.wait()

        descs = []
        nv = ROWS_PER_W // 16
        for k in range(KVOL):
            if k > 0:
                def off_body(j, carry, _off=k * NPAD, _base=k * nv):
                    idx_v[pl.ds((_base + j) * 16, 16)] += _off
                    return carry

                lax.fori_loop(0, nv, off_body, 0)
            descs.append(
                pltpu.async_copy(
                    y_hbm.at[idx_v.at[pl.ds(k * ROWS_PER_W, ROWS_PER_W)]],
                    acc_v,
                    sem,
                    add=True,
                )
            )
        for d in descs:
            d.wait()
        pltpu.sync_copy(acc_v, out_hbm.at[pl.ds(base, ROWS_PER_W)])

    return sc_k(y2d, nbr1d)


def kernel(in_feature, nbr_idx, weight, bias):
    w = weight.reshape(COUT, CIN, KVOL).transpose(2, 1, 0)
    w4 = jnp.einsum("tu,kcd->ktcud", jnp.eye(PACK, dtype=w.dtype), w)
    w4 = w4.reshape(KVOL, YW, YW)
    b128 = jnp.tile(bias.reshape(1, COUT), (1, PACK))
    x4 = jnp.pad(in_feature, ((0, NPAD - N), (0, 0))).reshape(NPAD // PACK, YW)
    nbr1d = jnp.pad(nbr_idx, ((0, 0), (0, NPAD - N))).reshape(-1)
    y = _tc_gemm(x4, w4, b128)
    out = _sc_gather_sum(y, nbr1d)
    return out[:N]
